# Initial kernel scaffold; baseline (speedup 1.0000x reference)
import jax
import jax.numpy as jnp
from jax.experimental import pallas as pl


def kernel(feat, e_feat, edge_index, W_apply, b_apply, edge_weight):
    raise NotImplementedError("write your pallas kernel here")


# speedup vs baseline: 3.7108x; 3.7108x over previous
import functools

import jax
import jax.numpy as jnp
from jax import lax
from jax.experimental import pallas as pl
from jax.experimental.pallas import tpu as pltpu
from jax.experimental.pallas import tpu_sc as plsc

N = 10000
E = 320000
D = 128
NT = 16
ALPHA = 10.0

C = 128
NCHUNK = E // C
NC = 2
NS = 16
NW = NC * NS
TPW = (NCHUNK + NW - 1) // NW
RBLK = 1000
NROW_STRIPE = N // NS


def _sc_mesh():
    return plsc.VectorSubcoreMesh(core_axis_name="c", subcore_axis_name="s")


@functools.partial(
    pl.kernel,
    mesh=_sc_mesh(),
    out_type=(
        jax.ShapeDtypeStruct((E,), jnp.int32),
        jax.ShapeDtypeStruct((N,), jnp.float32),
        jax.ShapeDtypeStruct((N,), jnp.float32),
    ),
    scratch_types=[
        pltpu.VMEM((16,), jnp.float32),
        pltpu.VMEM((C,), jnp.int32),
        pltpu.VMEM((C,), jnp.int32),
        pltpu.VMEM((C,), jnp.int32),
        pltpu.VMEM((C,), jnp.int32),
        pltpu.VMEM((C,), jnp.float32),
        pltpu.VMEM((N,), jnp.float32),
        pltpu.VMEM_SHARED((N,), jnp.float32),
    ],
)
def _sc_deg(ew_hbm, ef_hbm, src_hbm, dst_hbm, gidx_hbm, deg0_hbm, deg1_hbm,
            ew_v, ef_v, src_v, dst_v, gidx_v, c_v, degbuf_v, deg_sh):
    cid = lax.axis_index("c")
    sid = lax.axis_index("s")
    wid = sid * NC + cid

    pltpu.sync_copy(ew_hbm, ew_v)
    t = ew_v[...] * ALPHA
    tbl16 = jnp.where(t >= 0.0, t, t * 0.01)

    @pl.when(sid == 0)
    def _zero_deg():
        def zbody(i, carry):
            degbuf_v[pl.ds(i * 16, 16)] = jnp.zeros((16,), jnp.float32)
            return carry
        lax.fori_loop(0, N // 16, zbody, 0)
        pltpu.sync_copy(degbuf_v, deg_sh)
    plsc.subcore_barrier()

    def body(ti, carry):
        chunk = wid + ti * NW

        @pl.when(chunk < NCHUNK)
        def _chunk():
            base = chunk * C
            pltpu.sync_copy(ef_hbm.at[pl.ds(base, C)], ef_v)
            pltpu.sync_copy(src_hbm.at[pl.ds(base, C)], src_v)
            pltpu.sync_copy(dst_hbm.at[pl.ds(base, C)], dst_v)
            for g in range(C // 16):
                sl = pl.ds(g * 16, 16)
                k16 = (ef_v[sl] + (NT - 1)) & (NT - 1)
                gidx_v[sl] = k16 * N + src_v[sl]
                c_v[sl] = tbl16.at[k16].get(mode="promise_in_bounds")
            pltpu.sync_copy(gidx_v, gidx_hbm.at[pl.ds(base, C)])
            pltpu.sync_copy(c_v, deg_sh.at[dst_v], add=True)
        return carry
    lax.fori_loop(0, TPW, body, 0)
    plsc.subcore_barrier()

    @pl.when(sid == 0)
    def _write_deg():
        pltpu.sync_copy(deg_sh, degbuf_v)

        @pl.when(cid == 0)
        def _w0():
            pltpu.sync_copy(degbuf_v, deg0_hbm)

        @pl.when(cid == 1)
        def _w1():
            pltpu.sync_copy(degbuf_v, deg1_hbm)


@functools.partial(
    pl.kernel,
    mesh=_sc_mesh(),
    out_type=jax.ShapeDtypeStruct((NC, N, D), jnp.float32),
    scratch_types=[
        pltpu.VMEM((C,), jnp.int32),
        pltpu.VMEM((C,), jnp.int32),
        pltpu.VMEM((C, D), jnp.float32),
        pltpu.VMEM((200, D), jnp.float32),
        pltpu.VMEM_SHARED((N, D), jnp.float32),
        pltpu.SemaphoreType.DMA,
    ],
)
def _sc_msg(gidx_hbm, dst_hbm, h16_hbm, part_hbm,
            idx_v, dst_v, rows_v, bounce_v, acc_sh, sem):
    cid = lax.axis_index("c")
    sid = lax.axis_index("s")
    wid = sid * NC + cid

    def zbody(r, carry):
        for g in range(D // 16):
            bounce_v[r, pl.ds(g * 16, 16)] = jnp.zeros((16,), jnp.float32)
        return carry
    lax.fori_loop(0, 200, zbody, 0)

    @pl.when(sid < 10)
    def _zero_acc():
        for j in range(5):
            r0 = sid * 1000 + j * 200
            pltpu.sync_copy(bounce_v, acc_sh.at[pl.ds(r0, 200)])
    plsc.subcore_barrier()

    def body(ti, carry):
        chunk = wid + ti * NW

        @pl.when(chunk < NCHUNK)
        def _chunk():
            base = chunk * C
            pltpu.sync_copy(gidx_hbm.at[pl.ds(base, C)], idx_v)
            pltpu.sync_copy(dst_hbm.at[pl.ds(base, C)], dst_v)
            pltpu.async_copy(h16_hbm.at[idx_v], rows_v, sem).wait()
            pltpu.sync_copy(rows_v, acc_sh.at[dst_v], add=True)
        return carry
    lax.fori_loop(0, TPW, body, 0)
    plsc.subcore_barrier()

    @pl.when(sid < 10)
    def _write_part():
        for j in range(5):
            r0 = sid * 1000 + j * 200
            pltpu.sync_copy(acc_sh.at[pl.ds(r0, 200)], bounce_v)
            pltpu.sync_copy(bounce_v, part_hbm.at[cid, pl.ds(r0, 200)])


def _k2_body(ew_ref, deg_ref, feat_ref, out_ref):
    k = pl.program_id(0)
    t = ew_ref[...] * ALPHA
    tbl = jnp.where(t >= 0.0, t, t * 0.01)
    kk = lax.broadcasted_iota(jnp.int32, (NT, 1), 0)
    ew_k = jnp.sum(jnp.where(kk == k, tbl, 0.0))
    d = deg_ref[:, 0:1] + deg_ref[:, 1:2]
    norm = lax.rsqrt(jnp.maximum(d, 1.0))
    out_ref[...] = feat_ref[...] * norm * ew_k


def _k2_call(ew2, deg_t, feat):
    nblk = N // RBLK
    return pl.pallas_call(
        _k2_body,
        grid=(NT, nblk),
        in_specs=[
            pl.BlockSpec((NT, 1), lambda k, i: (0, 0)),
            pl.BlockSpec((RBLK, 2), lambda k, i: (i, 0)),
            pl.BlockSpec((RBLK, D), lambda k, i: (i, 0)),
        ],
        out_specs=pl.BlockSpec((RBLK, D), lambda k, i: (k * (N // RBLK) + i, 0)),
        out_shape=jax.ShapeDtypeStruct((NT * N, D), jnp.float32),
    )(ew2, deg_t, feat)


def _k4_body(part_ref, deg_ref, w_ref, b_ref, out_ref):
    p = part_ref[0] + part_ref[1]
    d = deg_ref[:, 0:1] + deg_ref[:, 1:2]
    norm = lax.rsqrt(jnp.maximum(d, 1.0))
    x = p * norm
    out_ref[...] = (
        jnp.dot(x, w_ref[...], preferred_element_type=jnp.float32) + b_ref[...]
    )


def _k4_call(part, deg_t, w, b2):
    return pl.pallas_call(
        _k4_body,
        grid=(N // RBLK,),
        in_specs=[
            pl.BlockSpec((NC, RBLK, D), lambda i: (0, i, 0)),
            pl.BlockSpec((RBLK, 2), lambda i: (i, 0)),
            pl.BlockSpec((D, D), lambda i: (0, 0)),
            pl.BlockSpec((1, D), lambda i: (0, 0)),
        ],
        out_specs=pl.BlockSpec((RBLK, D), lambda i: (i, 0)),
        out_shape=jax.ShapeDtypeStruct((N, D), jnp.float32),
    )(part, deg_t, w, b2)


def kernel(feat, e_feat, edge_index, W_apply, b_apply, edge_weight):
    src = edge_index[0]
    dst = edge_index[1]
    ew_flat = edge_weight.reshape(NT)

    gidx, deg0, deg1 = _sc_deg(ew_flat, e_feat, src, dst)
    deg_t = jnp.stack([deg0, deg1], axis=-1)
    h16 = _k2_call(edge_weight, deg_t, feat)
    part = _sc_msg(gidx, dst, h16)
    out = _k4_call(part, deg_t, W_apply, b_apply.reshape(1, D))
    return out
